# baseline (speedup 1.0000x reference)
import functools

import jax
import jax.numpy as jnp
from jax import lax
from jax.experimental import pallas as pl
from jax.experimental.pallas import tpu as pltpu
from jax.experimental.pallas import tpu_sc as plsc

_NC = 2
_NS = 16
_NW = _NC * _NS


def _argmin_body(x_ref, embed_ref, ind_ref, negx_ref):
    f = x_ref[...]
    c = embed_ref[...]
    ab = lax.dot_general(2.0 * f, c, (((1,), (1,)), ((), ())),
                         preferred_element_type=jnp.float32)
    f2 = jnp.sum(f * f, axis=1, keepdims=True)
    c2 = jnp.sum(c * c, axis=1)[None, :]
    dist = (f2 - ab) + c2
    m = jnp.min(dist, axis=1, keepdims=True)
    kidx = lax.broadcasted_iota(jnp.int32, dist.shape, 1)
    ind_ref[...] = jnp.min(jnp.where(dist <= m, kidx, dist.shape[1]), axis=1)
    negx_ref[...] = -f


def _sc_gather_body(ind_hbm, negx_hbm, embed_hbm, q_hbm, cd_hbm,
                    idx_v, rows_v, xb_v, sem_q, sem_c):
    wid = lax.axis_index("s") * _NC + lax.axis_index("c")
    n_rows = negx_hbm.shape[0]
    b = n_rows // _NW
    jrows = b // 128
    base = wid * b
    pltpu.sync_copy(ind_hbm.at[pl.ds(wid * jrows, jrows)], idx_v)
    pltpu.sync_copy(negx_hbm.at[pl.ds(base, b)], xb_v)
    copies = []
    for j in range(jrows):
        copies.append(pltpu.async_copy(
            embed_hbm.at[idx_v.at[j]], rows_v.at[pl.ds(j * 128, 128)], sem_q))
        copies.append(pltpu.async_copy(
            embed_hbm.at[idx_v.at[j]], xb_v.at[pl.ds(j * 128, 128)], sem_c,
            add=True))
    for cp in copies:
        cp.wait()
    pltpu.sync_copy(rows_v, q_hbm.at[pl.ds(base, b)])
    pltpu.sync_copy(xb_v, cd_hbm.at[pl.ds(base, b)])


@jax.jit
def kernel(x, embed):
    d = x.shape[-1]
    k = embed.shape[0]
    flat = x.reshape(-1, d)
    n = flat.shape[0]
    tn = 2048
    ind, negx = pl.pallas_call(
        _argmin_body,
        grid=(n // tn,),
        in_specs=[
            pl.BlockSpec((tn, d), lambda i: (i, 0)),
            pl.BlockSpec((k, d), lambda i: (0, 0)),
        ],
        out_specs=[
            pl.BlockSpec((tn,), lambda i: (i,)),
            pl.BlockSpec((tn, d), lambda i: (i, 0)),
        ],
        out_shape=[
            jax.ShapeDtypeStruct((n,), jnp.int32),
            jax.ShapeDtypeStruct((n, d), jnp.float32),
        ],
    )(flat, embed)

    b = n // _NW
    mesh = plsc.VectorSubcoreMesh(core_axis_name="c", subcore_axis_name="s")
    sc_gather = functools.partial(
        pl.kernel,
        out_type=[
            jax.ShapeDtypeStruct((n, d), jnp.float32),
            jax.ShapeDtypeStruct((n, d), jnp.float32),
        ],
        mesh=mesh,
        compiler_params=pltpu.CompilerParams(use_tc_tiling_on_sc=False),
        scratch_types=[
            pltpu.VMEM((b // 128, 128), jnp.int32),
            pltpu.VMEM((b, d), jnp.float32),
            pltpu.VMEM((b, d), jnp.float32),
            pltpu.SemaphoreType.DMA,
            pltpu.SemaphoreType.DMA,
        ],
    )(_sc_gather_body)
    q, cd = sc_gather(ind.reshape(n // 128, 128), negx, embed)
    return (q, ind, cd)


# speedup vs baseline: 3.3035x; 3.3035x over previous
import jax
import jax.numpy as jnp
from jax import lax
from jax.experimental import pallas as pl


def _vq_body(xt_ref, embed_ref, embed_t_ref, ind_ref, qt_ref, cdt_ref):
    ft = xt_ref[0]
    c = embed_ref[...]
    ct = embed_t_ref[...]
    ab_t = lax.dot_general(c, 2.0 * ft, (((1,), (0,)), ((), ())),
                           preferred_element_type=jnp.float32)
    f2 = jnp.sum(ft * ft, axis=0, keepdims=True)
    c2 = jnp.sum(c * c, axis=1)[:, None]
    dist_t = (f2 - ab_t) + c2
    m = jnp.min(dist_t, axis=0, keepdims=True)
    kidx = lax.broadcasted_iota(jnp.int32, dist_t.shape, 0)
    ind = jnp.min(jnp.where(dist_t <= m, kidx, dist_t.shape[0]), axis=0)
    ind_ref[...] = ind
    onehot_t = (kidx == ind[None, :]).astype(jnp.float32)
    qt = lax.dot_general(ct, onehot_t, (((1,), (0,)), ((), ())),
                         preferred_element_type=jnp.float32)
    qt_ref[...] = qt
    cdt_ref[...] = qt - ft


@jax.jit
def kernel(x, embed):
    d = x.shape[-1]
    k = embed.shape[0]
    n = x.shape[0] * x.shape[1]
    tn = x.shape[1]
    xt = jnp.transpose(x, (0, 2, 1))
    embed_t = embed.T
    ind, qt, cdt = pl.pallas_call(
        _vq_body,
        grid=(n // tn,),
        in_specs=[
            pl.BlockSpec((1, d, tn), lambda i: (i, 0, 0)),
            pl.BlockSpec((k, d), lambda i: (0, 0)),
            pl.BlockSpec((d, k), lambda i: (0, 0)),
        ],
        out_specs=[
            pl.BlockSpec((tn,), lambda i: (i,)),
            pl.BlockSpec((d, tn), lambda i: (0, i)),
            pl.BlockSpec((d, tn), lambda i: (0, i)),
        ],
        out_shape=[
            jax.ShapeDtypeStruct((n,), jnp.int32),
            jax.ShapeDtypeStruct((d, n), jnp.float32),
            jax.ShapeDtypeStruct((d, n), jnp.float32),
        ],
    )(xt, embed, embed_t)
    return (qt.T, ind, cdt.T)
